# baseline (speedup 1.0000x reference)
import functools

import jax
import jax.numpy as jnp
from jax import lax
from jax.experimental import pallas as pl
from jax.experimental.pallas import tpu as pltpu
from jax.experimental.pallas import tpu_sc as plsc

N = 10000
E = 320000
NFEAT = 128
NHID = 128
NCLASS = 10

BLK = 1000
NSUB = 16
K = 128
NPAD = 10240
RPS = NPAD // NSUB
NIB = 4

_MESH = plsc.VectorSubcoreMesh(core_axis_name="c", subcore_axis_name="s")


def _mm_in(x, W):
    def body(x_ref, w_ref, o_ref):
        o_ref[...] = jnp.dot(x_ref[...], w_ref[...],
                             preferred_element_type=jnp.float32)

    return pl.pallas_call(
        body,
        grid=(N // BLK,),
        in_specs=[
            pl.BlockSpec((BLK, x.shape[1]), lambda i: (i, 0)),
            pl.BlockSpec(W.shape, lambda i: (0, 0)),
        ],
        out_specs=pl.BlockSpec((BLK, W.shape[1]), lambda i: (i, 0)),
        out_shape=jax.ShapeDtypeStruct((N, W.shape[1]), jnp.float32),
    )(x, W)


def _relu_sum(part, b):
    F = part.shape[2]

    def body(p_ref, b_ref, o_ref):
        o_ref[...] = jnp.maximum(p_ref[0] + p_ref[1] + b_ref[...], 0.0)

    return pl.pallas_call(
        body,
        grid=(N // BLK,),
        in_specs=[
            pl.BlockSpec((2, BLK, F), lambda i: (0, i, 0)),
            pl.BlockSpec((1, F), lambda i: (0, 0)),
        ],
        out_specs=pl.BlockSpec((BLK, F), lambda i: (i, 0)),
        out_shape=jax.ShapeDtypeStruct((N, F), jnp.float32),
    )(part, b.reshape(1, F))


def _head(part, W2, b2, W3, b3, W4, b4):
    F = part.shape[2]
    F2 = W2.shape[1]
    nsteps = N // BLK

    def body(p_ref, w2_ref, b2_ref, w3_ref, b3_ref, w4_ref, b4_ref,
             o_ref, acc_ref):
        i = pl.program_id(0)

        @pl.when(i == 0)
        def _():
            acc_ref[...] = jnp.zeros_like(acc_ref)

        a = p_ref[0] + p_ref[1]
        h = jnp.dot(a, w2_ref[...], preferred_element_type=jnp.float32)
        h = jnp.maximum(h + b2_ref[...], 0.0)
        acc_ref[...] += jnp.sum(h, axis=0, keepdims=True)

        @pl.when(i == nsteps - 1)
        def _():
            g = jnp.maximum(acc_ref[...] / N, 0.0)
            g = jnp.maximum(
                jnp.dot(g, w3_ref[...], preferred_element_type=jnp.float32)
                + b3_ref[...], 0.0)
            logits = (jnp.dot(g, w4_ref[...], preferred_element_type=jnp.float32)
                      + b4_ref[...])
            m = jnp.max(logits)
            z = logits - m
            o_ref[...] = z - jnp.log(jnp.sum(jnp.exp(z)))

    return pl.pallas_call(
        body,
        grid=(nsteps,),
        in_specs=[
            pl.BlockSpec((2, BLK, F), lambda i: (0, i, 0)),
            pl.BlockSpec(W2.shape, lambda i: (0, 0)),
            pl.BlockSpec((1, F2), lambda i: (0, 0)),
            pl.BlockSpec(W3.shape, lambda i: (0, 0)),
            pl.BlockSpec((1, W3.shape[1]), lambda i: (0, 0)),
            pl.BlockSpec(W4.shape, lambda i: (0, 0)),
            pl.BlockSpec((1, NCLASS), lambda i: (0, 0)),
        ],
        out_specs=pl.BlockSpec((1, NCLASS), lambda i: (0, 0)),
        out_shape=jax.ShapeDtypeStruct((1, NCLASS), jnp.float32),
        scratch_shapes=[pltpu.VMEM((1, F2), jnp.float32)],
    )(part, W2, b2.reshape(1, F2), W3, b3.reshape(1, W3.shape[1]), W4,
      b4.reshape(1, NCLASS))


def _gather_scatter_loop(sup, ic, ibufs, isems, bufs, gsems, acc, nchunk):
    for j in range(NIB):
        pltpu.make_async_copy(ic.at[j], ibufs[j], isems[j]).start()
    for b in range(2):
        pltpu.make_async_copy(ic.at[b], ibufs[b], isems[b]).wait()
        pltpu.make_async_copy(sup.at[ibufs[b].at[0]], bufs[b], gsems[b]).start()

    def body(ch, u):
        b = u % 2
        pltpu.make_async_copy(sup.at[ibufs[u].at[0]], bufs[b], gsems[b]).wait()
        pltpu.sync_copy(bufs[b], acc.at[ibufs[u].at[1]], add=True)

        @pl.when(ch + NIB < nchunk)
        def _():
            pltpu.make_async_copy(ic.at[ch + NIB], ibufs[u], isems[u]).start()

        @pl.when(ch + 2 < nchunk)
        def _():
            u2 = (u + 2) % NIB
            pltpu.make_async_copy(ic.at[ch + 2], ibufs[u2], isems[u2]).wait()
            pltpu.make_async_copy(
                sup.at[ibufs[u2].at[0]], bufs[b], gsems[b]).start()

    @pl.loop(0, nchunk, step=NIB)
    def _(i):
        for u in range(NIB):
            body(i + u, u)


def _zero_acc(buf, acc, row0):
    kk, ff = buf.shape

    @pl.loop(0, kk)
    def _(r):
        for j in range(ff // 16):
            buf[r, pl.ds(16 * j, 16)] = jnp.zeros((16,), jnp.float32)

    @pl.loop(0, RPS // kk)
    def _(t):
        pltpu.sync_copy(buf, acc.at[pl.ds(row0 + t * kk, kk)])


def _sc_agg(support, idx, nchunk):
    F = support.shape[1]

    @functools.partial(
        pl.kernel,
        out_type=jax.ShapeDtypeStruct((2, NPAD, F), jnp.float32),
        mesh=_MESH,
        scratch_types=[
            pltpu.VMEM((2, K), jnp.int32),
            pltpu.VMEM((2, K), jnp.int32),
            pltpu.VMEM((2, K), jnp.int32),
            pltpu.VMEM((2, K), jnp.int32),
            pltpu.VMEM((K, F), jnp.float32),
            pltpu.VMEM((K, F), jnp.float32),
            pltpu.VMEM_SHARED((NPAD, F), jnp.float32),
            pltpu.SemaphoreType.DMA,
            pltpu.SemaphoreType.DMA,
            pltpu.SemaphoreType.DMA,
            pltpu.SemaphoreType.DMA,
            pltpu.SemaphoreType.DMA,
            pltpu.SemaphoreType.DMA,
        ],
    )
    def k(sup_hbm, ic_hbm, out_hbm,
          ib0, ib1, ib2, ib3, bufa, bufb, acc,
          is0, is1, is2, is3, gsa, gsb):
        c = lax.axis_index("c")
        s = lax.axis_index("s")
        row0 = s * RPS
        _zero_acc(bufa, acc, row0)
        plsc.subcore_barrier()

        ic = ic_hbm.at[c].at[s]
        _gather_scatter_loop(sup_hbm, ic, (ib0, ib1, ib2, ib3),
                             (is0, is1, is2, is3), (bufa, bufb), (gsa, gsb),
                             acc, nchunk)

        plsc.subcore_barrier()
        pltpu.sync_copy(acc.at[pl.ds(row0, RPS)],
                        out_hbm.at[c].at[pl.ds(row0, RPS)])

    return k(support, idx)


def kernel(x, edge_index, W1, b1, W2, b2, W3, b3, W4, b4):
    src = edge_index[0].astype(jnp.int32)
    dst = edge_index[1].astype(jnp.int32)
    per = E // 2 // NSUB
    tgt = -(-per // (NIB * K)) * (NIB * K)
    nch = tgt // K
    s_r = jnp.pad(src.reshape(2, NSUB, per), ((0, 0), (0, 0), (0, tgt - per)))
    d_r = jnp.pad(dst.reshape(2, NSUB, per), ((0, 0), (0, 0), (0, tgt - per)),
                  constant_values=N)
    ic = jnp.stack([s_r.reshape(2, NSUB, nch, K),
                    d_r.reshape(2, NSUB, nch, K)], axis=3)

    support1 = _mm_in(x, W1)
    part1 = _sc_agg(support1, ic, nch)
    h1 = _relu_sum(part1, b1)
    part2 = _sc_agg(h1, ic, nch)
    out = _head(part2, W2, b2, W3, b3, W4, b4)
    return out.reshape(NCLASS)


# speedup vs baseline: 2.7729x; 2.7729x over previous
import functools

import jax
import jax.numpy as jnp
from jax import lax
from jax.experimental import pallas as pl
from jax.experimental.pallas import tpu as pltpu
from jax.experimental.pallas import tpu_sc as plsc

N = 10000
E = 320000
NFEAT = 128
NHID = 128
NCLASS = 10

BLK = 1000
NSUB = 16
K = 80
NPAD = 10240
RPS = NPAD // NSUB
NIB = 4

_MESH = plsc.VectorSubcoreMesh(core_axis_name="c", subcore_axis_name="s")


def _mm_in(x, W):
    def body(x_ref, w_ref, o_ref):
        o_ref[...] = jnp.dot(x_ref[...], w_ref[...],
                             preferred_element_type=jnp.float32)

    return pl.pallas_call(
        body,
        grid=(N // BLK,),
        in_specs=[
            pl.BlockSpec((BLK, x.shape[1]), lambda i: (i, 0)),
            pl.BlockSpec(W.shape, lambda i: (0, 0)),
        ],
        out_specs=pl.BlockSpec((BLK, W.shape[1]), lambda i: (i, 0)),
        out_shape=jax.ShapeDtypeStruct((N, W.shape[1]), jnp.float32),
    )(x, W)


def _relu_sum(part, b):
    F = part.shape[2]

    def body(p_ref, b_ref, o_ref):
        o_ref[...] = jnp.maximum(p_ref[0] + p_ref[1] + b_ref[...], 0.0)

    return pl.pallas_call(
        body,
        grid=(N // BLK,),
        in_specs=[
            pl.BlockSpec((2, BLK, F), lambda i: (0, i, 0)),
            pl.BlockSpec((1, F), lambda i: (0, 0)),
        ],
        out_specs=pl.BlockSpec((BLK, F), lambda i: (i, 0)),
        out_shape=jax.ShapeDtypeStruct((N, F), jnp.float32),
    )(part, b.reshape(1, F))


def _head(part, W2, b2, W3, b3, W4, b4):
    F = part.shape[2]
    F2 = W2.shape[1]
    nsteps = N // BLK

    def body(p_ref, w2_ref, b2_ref, w3_ref, b3_ref, w4_ref, b4_ref,
             o_ref, acc_ref):
        i = pl.program_id(0)

        @pl.when(i == 0)
        def _():
            acc_ref[...] = jnp.zeros_like(acc_ref)

        a = p_ref[0] + p_ref[1]
        h = jnp.dot(a, w2_ref[...], preferred_element_type=jnp.float32)
        h = jnp.maximum(h + b2_ref[...], 0.0)
        acc_ref[...] += jnp.sum(h, axis=0, keepdims=True)

        @pl.when(i == nsteps - 1)
        def _():
            g = jnp.maximum(acc_ref[...] / N, 0.0)
            g = jnp.maximum(
                jnp.dot(g, w3_ref[...], preferred_element_type=jnp.float32)
                + b3_ref[...], 0.0)
            logits = (jnp.dot(g, w4_ref[...], preferred_element_type=jnp.float32)
                      + b4_ref[...])
            m = jnp.max(logits)
            z = logits - m
            o_ref[...] = z - jnp.log(jnp.sum(jnp.exp(z)))

    return pl.pallas_call(
        body,
        grid=(nsteps,),
        in_specs=[
            pl.BlockSpec((2, BLK, F), lambda i: (0, i, 0)),
            pl.BlockSpec(W2.shape, lambda i: (0, 0)),
            pl.BlockSpec((1, F2), lambda i: (0, 0)),
            pl.BlockSpec(W3.shape, lambda i: (0, 0)),
            pl.BlockSpec((1, W3.shape[1]), lambda i: (0, 0)),
            pl.BlockSpec(W4.shape, lambda i: (0, 0)),
            pl.BlockSpec((1, NCLASS), lambda i: (0, 0)),
        ],
        out_specs=pl.BlockSpec((1, NCLASS), lambda i: (0, 0)),
        out_shape=jax.ShapeDtypeStruct((1, NCLASS), jnp.float32),
        scratch_shapes=[pltpu.VMEM((1, F2), jnp.float32)],
    )(part, W2, b2.reshape(1, F2), W3, b3.reshape(1, W3.shape[1]), W4,
      b4.reshape(1, NCLASS))


def _gather_scatter_loop(sup, ic, ibufs, isems, bufs, gsems, acc, nchunk):
    for j in range(NIB):
        pltpu.make_async_copy(ic.at[j], ibufs[j], isems[j]).start()
    for b in range(2):
        pltpu.make_async_copy(ic.at[b], ibufs[b], isems[b]).wait()
        pltpu.make_async_copy(sup.at[ibufs[b].at[0]], bufs[b], gsems[b]).start()

    def body(ch, u):
        b = u % 2
        pltpu.make_async_copy(sup.at[ibufs[u].at[0]], bufs[b], gsems[b]).wait()
        pltpu.sync_copy(bufs[b], acc.at[ibufs[u].at[1]], add=True)

        @pl.when(ch + NIB < nchunk)
        def _():
            pltpu.make_async_copy(ic.at[ch + NIB], ibufs[u], isems[u]).start()

        @pl.when(ch + 2 < nchunk)
        def _():
            u2 = (u + 2) % NIB
            pltpu.make_async_copy(ic.at[ch + 2], ibufs[u2], isems[u2]).wait()
            pltpu.make_async_copy(
                sup.at[ibufs[u2].at[0]], bufs[b], gsems[b]).start()

    nquad = (nchunk // NIB) * NIB

    @pl.loop(0, nquad, step=NIB)
    def _(i):
        for u in range(NIB):
            body(i + u, u)

    for ch in range(nquad, nchunk):
        u = ch % NIB
        b = u % 2
        pltpu.make_async_copy(sup.at[ibufs[u].at[0]], bufs[b], gsems[b]).wait()
        pltpu.sync_copy(bufs[b], acc.at[ibufs[u].at[1]], add=True)


def _zero_acc(buf, acc, row0):
    kk, ff = buf.shape

    @pl.loop(0, kk)
    def _(r):
        for j in range(ff // 16):
            buf[r, pl.ds(16 * j, 16)] = jnp.zeros((16,), jnp.float32)

    @pl.loop(0, RPS // kk)
    def _(t):
        pltpu.sync_copy(buf, acc.at[pl.ds(row0 + t * kk, kk)])


def _sc_agg(support, idx, nchunk):
    F = support.shape[1]

    @functools.partial(
        pl.kernel,
        out_type=jax.ShapeDtypeStruct((2, NPAD, F), jnp.float32),
        mesh=_MESH,
        scratch_types=[
            pltpu.VMEM((2, K), jnp.int32),
            pltpu.VMEM((2, K), jnp.int32),
            pltpu.VMEM((2, K), jnp.int32),
            pltpu.VMEM((2, K), jnp.int32),
            pltpu.VMEM((K, F), jnp.float32),
            pltpu.VMEM((K, F), jnp.float32),
            pltpu.VMEM_SHARED((NPAD, F), jnp.float32),
            pltpu.SemaphoreType.DMA,
            pltpu.SemaphoreType.DMA,
            pltpu.SemaphoreType.DMA,
            pltpu.SemaphoreType.DMA,
            pltpu.SemaphoreType.DMA,
            pltpu.SemaphoreType.DMA,
        ],
    )
    def k(sup_hbm, ic_hbm, out_hbm,
          ib0, ib1, ib2, ib3, bufa, bufb, acc,
          is0, is1, is2, is3, gsa, gsb):
        c = lax.axis_index("c")
        s = lax.axis_index("s")
        row0 = s * RPS
        _zero_acc(bufa, acc, row0)
        plsc.subcore_barrier()

        ic = ic_hbm.at[c].at[s]
        _gather_scatter_loop(sup_hbm, ic, (ib0, ib1, ib2, ib3),
                             (is0, is1, is2, is3), (bufa, bufb), (gsa, gsb),
                             acc, nchunk)

        plsc.subcore_barrier()
        pltpu.sync_copy(acc.at[pl.ds(row0, RPS)],
                        out_hbm.at[c].at[pl.ds(row0, RPS)])

    return k(support, idx)


def kernel(x, edge_index, W1, b1, W2, b2, W3, b3, W4, b4):
    src = edge_index[0].astype(jnp.int32)
    dst = edge_index[1].astype(jnp.int32)
    per = E // 2 // NSUB
    nch = per // K
    ic = jnp.stack([src.reshape(2, NSUB, nch, K),
                    dst.reshape(2, NSUB, nch, K)], axis=3)

    support1 = _mm_in(x, W1)
    part1 = _sc_agg(support1, ic, nch)
    h1 = _relu_sum(part1, b1)
    part2 = _sc_agg(h1, ic, nch)
    out = _head(part2, W2, b2, W3, b3, W4, b4)
    return out.reshape(NCLASS)
